# baseline (speedup 1.0000x reference)
import jax
import jax.numpy as jnp
from jax import lax
from jax.experimental import pallas as pl
from jax.experimental.pallas import tpu as pltpu
from jax.experimental.pallas import tpu_sc as plsc

_B = 16384
_D = 32
_NC = 2
_NS = 16
_NW = _NC * _NS
_BPW = _B // _NW
_L = 16
_G = _BPW // _L


def _sc_body(u_hbm, i_hbm, uemb_hbm, iemb_hbm, out_hbm,
             uidx_v, iidx_v, urows_v, irows_v, out_v,
             sem_idx, sem_rows):
    wid = lax.axis_index("s") * _NC + lax.axis_index("c")
    base = wid * _BPW

    cp_u = pltpu.async_copy(u_hbm.at[pl.ds(base, _BPW)], uidx_v, sem_idx)
    cp_i = pltpu.async_copy(i_hbm.at[pl.ds(base, _BPW)], iidx_v, sem_idx)
    cp_u.wait()
    cp_i.wait()

    lane = lax.iota(jnp.int32, _L)

    def group_body(g, carry):
        descs = []
        for h in range(2):
            uvec = uidx_v[pl.ds(g * 2 * _L + h * _L, _L)]
            ivec = iidx_v[pl.ds(g * 2 * _L + h * _L, _L)]
            for j in range(_L):
                descs.append(pltpu.async_copy(
                    uemb_hbm.at[pl.ds(uvec[j], 1), :],
                    urows_v.at[pl.ds(h * _L + j, 1), :], sem_rows))
                descs.append(pltpu.async_copy(
                    iemb_hbm.at[pl.ds(ivec[j], 1), :],
                    irows_v.at[pl.ds(h * _L + j, 1), :], sem_rows))
        for dsc in descs:
            dsc.wait()

        for h in range(2):
            acc = jnp.zeros((_L,), jnp.float32)
            for jr in range(_L):
                r = h * _L + jr
                u0 = urows_v[r, pl.ds(0, _L)]
                u1 = urows_v[r, pl.ds(_L, _L)]
                i0 = irows_v[r, pl.ds(0, _L)]
                i1 = irows_v[r, pl.ds(_L, _L)]
                s = jnp.sum(u0 * i0 + u1 * i1)
                acc = jnp.where(lane == jr, s, acc)
            out_v[pl.ds(g * 2 * _L + h * _L, _L)] = acc
        return carry

    lax.fori_loop(0, _G // 2, group_body, 0)

    pltpu.sync_copy(out_v, out_hbm.at[pl.ds(base, _BPW)])


@jax.jit
def _bprmf_sc(u, i, user_emb, item_emb):
    mesh = plsc.VectorSubcoreMesh(core_axis_name="c", subcore_axis_name="s")
    run = pl.kernel(
        _sc_body,
        out_type=jax.ShapeDtypeStruct((_B,), jnp.float32),
        mesh=mesh,
        scratch_types=[
            pltpu.VMEM((_BPW,), jnp.int32),
            pltpu.VMEM((_BPW,), jnp.int32),
            pltpu.VMEM((2 * _L, _D), jnp.float32),
            pltpu.VMEM((2 * _L, _D), jnp.float32),
            pltpu.VMEM((_BPW,), jnp.float32),
            pltpu.SemaphoreType.DMA,
            pltpu.SemaphoreType.DMA,
        ],
        compiler_params=pltpu.CompilerParams(needs_layout_passes=False),
    )
    return run(u, i, user_emb, item_emb)


def kernel(u, i, user_emb, item_emb):
    return _bprmf_sc(u.astype(jnp.int32), i.astype(jnp.int32),
                     user_emb, item_emb)


# speedup vs baseline: 1.0026x; 1.0026x over previous
import jax
import jax.numpy as jnp
from jax import lax
from jax.experimental import pallas as pl
from jax.experimental.pallas import tpu as pltpu
from jax.experimental.pallas import tpu_sc as plsc

_B = 16384
_D = 32
_NC = 2
_NS = 16
_NW = _NC * _NS
_BPW = _B // _NW
_L = 16
_G = _BPW // _L


def _sc_body(u_hbm, i_hbm, uemb_hbm, iemb_hbm, out_hbm,
             uidx_v, iidx_v, urows_v, irows_v, out_v,
             sem_idx, *sems):
    wid = lax.axis_index("s") * _NC + lax.axis_index("c")
    base = wid * _BPW

    cp_u = pltpu.async_copy(u_hbm.at[pl.ds(base, _BPW)], uidx_v, sem_idx)
    cp_i = pltpu.async_copy(i_hbm.at[pl.ds(base, _BPW)], iidx_v, sem_idx)
    cp_u.wait()
    cp_i.wait()

    lane = lax.iota(jnp.int32, _L)

    def group_body(g, carry):
        descs = []
        for h in range(2):
            uvec = uidx_v[pl.ds(g * 2 * _L + h * _L, _L)]
            ivec = iidx_v[pl.ds(g * 2 * _L + h * _L, _L)]
            for j in range(_L):
                descs.append(pltpu.async_copy(
                    uemb_hbm.at[pl.ds(uvec[j], 1), :],
                    urows_v.at[pl.ds(h * _L + j, 1), :],
                    sems[(2 * j) % len(sems)]))
                descs.append(pltpu.async_copy(
                    iemb_hbm.at[pl.ds(ivec[j], 1), :],
                    irows_v.at[pl.ds(h * _L + j, 1), :],
                    sems[(2 * j + 1) % len(sems)]))
        for dsc in descs:
            dsc.wait()

        for h in range(2):
            acc = jnp.zeros((_L,), jnp.float32)
            for jr in range(_L):
                r = h * _L + jr
                u0 = urows_v[r, pl.ds(0, _L)]
                u1 = urows_v[r, pl.ds(_L, _L)]
                i0 = irows_v[r, pl.ds(0, _L)]
                i1 = irows_v[r, pl.ds(_L, _L)]
                s = jnp.sum(u0 * i0 + u1 * i1)
                acc = jnp.where(lane == jr, s, acc)
            out_v[pl.ds(g * 2 * _L + h * _L, _L)] = acc
        return carry

    lax.fori_loop(0, _G // 2, group_body, 0)

    pltpu.sync_copy(out_v, out_hbm.at[pl.ds(base, _BPW)])


@jax.jit
def _bprmf_sc(u, i, user_emb, item_emb):
    mesh = plsc.VectorSubcoreMesh(core_axis_name="c", subcore_axis_name="s")
    run = pl.kernel(
        _sc_body,
        out_type=jax.ShapeDtypeStruct((_B,), jnp.float32),
        mesh=mesh,
        scratch_types=[
            pltpu.VMEM((_BPW,), jnp.int32),
            pltpu.VMEM((_BPW,), jnp.int32),
            pltpu.VMEM((2 * _L, _D), jnp.float32),
            pltpu.VMEM((2 * _L, _D), jnp.float32),
            pltpu.VMEM((_BPW,), jnp.float32),
            pltpu.SemaphoreType.DMA,
        ] + [pltpu.SemaphoreType.DMA] * 8,
        compiler_params=pltpu.CompilerParams(needs_layout_passes=False),
    )
    return run(u, i, user_emb, item_emb)


def kernel(u, i, user_emb, item_emb):
    return _bprmf_sc(u.astype(jnp.int32), i.astype(jnp.int32),
                     user_emb, item_emb)
